# Initial kernel scaffold; baseline (speedup 1.0000x reference)
import jax
import jax.numpy as jnp
from jax.experimental import pallas as pl


def kernel(xyz_coarse, feat_coarse, xyz_fine, feat_skip, W1, b1, g1, be1, W2, b2, g2, be2):
    raise NotImplementedError("write your pallas kernel here")


# speedup vs baseline: 9.1341x; 9.1341x over previous
import functools

import jax
import jax.numpy as jnp
from jax import lax
from jax.experimental import pallas as pl
from jax.experimental.pallas import tpu as pltpu
from jax.experimental.pallas import tpu_sc as plsc

G = 32
EPS_GN = 1e-5
K = 3

NUM_SC = 2
NUM_SUBCORES = 16
NW = NUM_SC * NUM_SUBCORES

TN = 512
BIG = 3.0e38


def _knn_body(nc, xf_ref, xc_ref, idx_ref, w_ref):
    b = pl.program_id(0)
    xf = xf_ref[0]
    xc = xc_ref[0]
    sq_f = jnp.sum(xf * xf, axis=1, keepdims=True)
    sq_c = jnp.sum(xc * xc, axis=0, keepdims=True)
    xfb = xf.astype(jnp.bfloat16)
    xcb = xc.astype(jnp.bfloat16)
    dot = jnp.dot(xfb, xcb, preferred_element_type=jnp.float32)
    d2 = sq_f + sq_c - 2.0 * dot
    d = jnp.sqrt(jnp.maximum(d2, 0.0))
    iota = lax.broadcasted_iota(jnp.int32, (TN, nc), 1)
    cur = d
    idxs, vals = [], []
    for _ in range(K):
        mn = jnp.min(cur, axis=1, keepdims=True)
        idxk = jnp.min(jnp.where(cur == mn, iota, nc), axis=1, keepdims=True)
        idxs.append(idxk)
        vals.append(mn)
        cur = jnp.where(iota == idxk, BIG, cur)
    d3 = jnp.concatenate(vals, axis=1)
    winv = 1.0 / (d3 + 1e-12)
    w = winv / jnp.sum(winv, axis=1, keepdims=True)
    lane3 = lax.broadcasted_iota(jnp.int32, (TN, K), 1)
    onehot0 = (lane3 == 0).astype(jnp.float32)
    w = jnp.where(d3[:, 0:1] <= 1e-12, onehot0, w)
    idx_ref[0] = jnp.concatenate(idxs, axis=1) + b * nc
    w_ref[0] = jnp.concatenate(
        [jnp.broadcast_to(w[:, k:k + 1], (TN, 16)) for k in range(K)], axis=1)


def _knn(xyz_fine, xc_t):
    b, nf, _ = xyz_fine.shape
    nc = xc_t.shape[2]
    return pl.pallas_call(
        functools.partial(_knn_body, nc),
        grid=(b, nf // TN),
        in_specs=[
            pl.BlockSpec((1, TN, 3), lambda i, j: (i, j, 0)),
            pl.BlockSpec((1, 3, nc), lambda i, j: (i, 0, 0)),
        ],
        out_specs=[
            pl.BlockSpec((1, TN, K), lambda i, j: (i, j, 0)),
            pl.BlockSpec((1, TN, 16 * K), lambda i, j: (i, j, 0)),
        ],
        out_shape=[
            jax.ShapeDtypeStruct((b, nf, K), jnp.int32),
            jax.ShapeDtypeStruct((b, nf, 16 * K), jnp.float32),
        ],
    )(xyz_fine, xc_t)


CP = 32
CR = K * CP


def _sc_interp_body(cc, p_per_w, table, idxf, wf, out, idx_v, w_v, rows_v,
                    out_v, sem):
    wid = lax.axis_index("s") * NUM_SC + lax.axis_index("c")
    base_p0 = wid * p_per_w

    def chunk(g, carry):
        base_p = base_p0 + g * CP
        base_r = K * base_p
        pltpu.sync_copy(idxf.at[pl.ds(base_r, CR)], idx_v)
        pltpu.sync_copy(wf.at[pl.ds(base_p, CP)], w_v)
        pltpu.async_copy(table.at[idx_v], rows_v, sem).wait()
        for p in range(CP):
            w0 = w_v[p, pl.ds(0, 16)]
            w1 = w_v[p, pl.ds(16, 16)]
            w2 = w_v[p, pl.ds(32, 16)]

            def col(j, _, p=p, w0=w0, w1=w1, w2=w2):
                s = pl.ds(j * 16, 16)
                out_v[p, s] = (w0 * rows_v[K * p, s] + w1 * rows_v[K * p + 1, s]
                               + w2 * rows_v[K * p + 2, s])
                return 0

            lax.fori_loop(0, cc // 16, col, 0)
        pltpu.sync_copy(out_v, out.at[pl.ds(base_p, CP)])
        return carry

    lax.fori_loop(0, p_per_w // CP, chunk, 0)


def _sc_interp(table, idx_flat, w_bcast):
    rows, cc = table.shape
    pts = idx_flat.shape[0] // K
    p_per_w = pts // NW
    mesh = plsc.VectorSubcoreMesh(core_axis_name="c", subcore_axis_name="s")
    f = functools.partial(
        pl.kernel,
        mesh=mesh,
        out_type=jax.ShapeDtypeStruct((pts, cc), jnp.float32),
        scratch_types=[
            pltpu.VMEM((CR,), jnp.int32),
            pltpu.VMEM((CP, 16 * K), jnp.float32),
            pltpu.VMEM((CR, cc), jnp.float32),
            pltpu.VMEM((CP, cc), jnp.float32),
            pltpu.SemaphoreType.DMA,
        ],
    )(functools.partial(_sc_interp_body, cc, p_per_w))
    return f(table, idx_flat, w_bcast)


CPG = 16


def _gn_relu(h, gamma, beta):
    n, out_ch = h.shape
    ng = out_ch // CPG
    m = (lax.broadcasted_iota(jnp.int32, (out_ch, ng), 0) // CPG ==
         lax.broadcasted_iota(jnp.int32, (out_ch, ng), 1)).astype(jnp.float32)
    mt = (lax.broadcasted_iota(jnp.int32, (ng, out_ch), 0) ==
          lax.broadcasted_iota(jnp.int32, (ng, out_ch), 1) // CPG
          ).astype(jnp.float32)
    s1 = jnp.sum(h, axis=0, keepdims=True)
    s2 = jnp.sum(h * h, axis=0, keepdims=True)
    cnt = jnp.float32(n * CPG)
    mu = jnp.dot(s1, m, preferred_element_type=jnp.float32) / cnt
    e2 = jnp.dot(s2, m, preferred_element_type=jnp.float32) / cnt
    inv = lax.rsqrt(e2 - mu * mu + EPS_GN)
    mu_c = jnp.dot(mu, mt, preferred_element_type=jnp.float32)
    inv_c = jnp.dot(inv, mt, preferred_element_type=jnp.float32)
    return jnp.maximum((h - mu_c) * inv_c * gamma + beta, 0.0)


TO = 128


def _layer1_body(x1_ref, x2_ref, w1a_ref, w1b_ref, b1_ref, g1_ref, be1_ref,
                 out_ref):
    h = jnp.dot(x1_ref[0], w1a_ref[...], preferred_element_type=jnp.float32)
    h = h + jnp.dot(x2_ref[0], w1b_ref[...], preferred_element_type=jnp.float32)
    out_ref[0] = _gn_relu(h + b1_ref[...], g1_ref[...], be1_ref[...])


def _layer2_body(x_ref, w2_ref, b2_ref, g2_ref, be2_ref, out_ref):
    h = jnp.dot(x_ref[0], w2_ref[...], preferred_element_type=jnp.float32)
    out_ref[0] = _gn_relu(h + b2_ref[...], g2_ref[...], be2_ref[...])


def _mlp(interp, skip, w1a_t, w1b_t, w2_t, b1, g1, be1, b2, g2, be2):
    b, nf, cc = interp.shape
    cs = skip.shape[2]
    out_ch = w2_t.shape[1]
    vec = pl.BlockSpec((1, TO), lambda i, o: (0, o))
    h1 = pl.pallas_call(
        _layer1_body,
        grid=(b, out_ch // TO),
        in_specs=[
            pl.BlockSpec((1, nf, cc), lambda i, o: (i, 0, 0)),
            pl.BlockSpec((1, nf, cs), lambda i, o: (i, 0, 0)),
            pl.BlockSpec((cc, TO), lambda i, o: (0, o)),
            pl.BlockSpec((cs, TO), lambda i, o: (0, o)),
            vec, vec, vec,
        ],
        out_specs=pl.BlockSpec((1, nf, TO), lambda i, o: (i, 0, o)),
        out_shape=jax.ShapeDtypeStruct((b, nf, out_ch), jnp.float32),
    )(interp, skip, w1a_t, w1b_t, b1, g1, be1)
    return pl.pallas_call(
        _layer2_body,
        grid=(b, out_ch // TO),
        in_specs=[
            pl.BlockSpec((1, nf, out_ch), lambda i, o: (i, 0, 0)),
            pl.BlockSpec((out_ch, TO), lambda i, o: (0, o)),
            vec, vec, vec,
        ],
        out_specs=pl.BlockSpec((1, nf, TO), lambda i, o: (i, 0, o)),
        out_shape=jax.ShapeDtypeStruct((b, nf, out_ch), jnp.float32),
    )(h1, w2_t, b2, g2, be2)


def kernel(xyz_coarse, feat_coarse, xyz_fine, feat_skip, W1, b1, g1, be1,
           W2, b2, g2, be2):
    b, nc, cc = feat_coarse.shape
    xc_t = jnp.swapaxes(xyz_coarse, 1, 2)
    idx, w = _knn(xyz_fine, xc_t)
    table = feat_coarse.reshape(b * nc, cc)
    interp = _sc_interp(table, idx.reshape(-1), w.reshape(-1, 16 * K))
    interp = interp.reshape(b, xyz_fine.shape[1], cc)
    w1a_t = W1[:, :cc].T
    w1b_t = W1[:, cc:].T
    w2_t = W2.T
    return _mlp(interp, feat_skip, w1a_t, w1b_t, w2_t,
                b1[None], g1[None], be1[None], b2[None], g2[None], be2[None])


# speedup vs baseline: 10.6444x; 1.1653x over previous
import functools

import jax
import jax.numpy as jnp
from jax import lax
from jax.experimental import pallas as pl
from jax.experimental.pallas import tpu as pltpu
from jax.experimental.pallas import tpu_sc as plsc

G = 32
EPS_GN = 1e-5
K = 3

NUM_SC = 2
NUM_SUBCORES = 16
NW = NUM_SC * NUM_SUBCORES

TN = 512
BIG = 3.0e38


def _knn_body(nc, xf_ref, xc_ref, idx_ref, w_ref):
    b = pl.program_id(0)
    xf = xf_ref[0]
    xc = xc_ref[0]
    sq_f = jnp.sum(xf * xf, axis=1, keepdims=True)
    sq_c = jnp.sum(xc * xc, axis=0, keepdims=True)
    xfb = xf.astype(jnp.bfloat16)
    xcb = xc.astype(jnp.bfloat16)
    dot = jnp.dot(xfb, xcb, preferred_element_type=jnp.float32)
    d2 = sq_f + sq_c - 2.0 * dot
    d = jnp.sqrt(jnp.maximum(d2, 0.0))
    iota = lax.broadcasted_iota(jnp.int32, (TN, nc), 1)
    cur = d
    idxs, vals = [], []
    for _ in range(K):
        mn = jnp.min(cur, axis=1, keepdims=True)
        idxk = jnp.min(jnp.where(cur == mn, iota, nc), axis=1, keepdims=True)
        idxs.append(idxk)
        vals.append(mn)
        cur = jnp.where(iota == idxk, BIG, cur)
    d3 = jnp.concatenate(vals, axis=1)
    winv = 1.0 / (d3 + 1e-12)
    w = winv / jnp.sum(winv, axis=1, keepdims=True)
    lane3 = lax.broadcasted_iota(jnp.int32, (TN, K), 1)
    onehot0 = (lane3 == 0).astype(jnp.float32)
    w = jnp.where(d3[:, 0:1] <= 1e-12, onehot0, w)
    idx_ref[0] = jnp.concatenate(idxs, axis=1) + b * nc
    w_ref[0] = jnp.concatenate(
        [jnp.broadcast_to(w[:, k:k + 1], (TN, 16)) for k in range(K)], axis=1)


def _knn(xyz_fine, xc_t):
    b, nf, _ = xyz_fine.shape
    nc = xc_t.shape[2]
    return pl.pallas_call(
        functools.partial(_knn_body, nc),
        grid=(b, nf // TN),
        in_specs=[
            pl.BlockSpec((1, TN, 3), lambda i, j: (i, j, 0)),
            pl.BlockSpec((1, 3, nc), lambda i, j: (i, 0, 0)),
        ],
        out_specs=[
            pl.BlockSpec((1, TN, K), lambda i, j: (i, j, 0)),
            pl.BlockSpec((1, TN, 16 * K), lambda i, j: (i, j, 0)),
        ],
        out_shape=[
            jax.ShapeDtypeStruct((b, nf, K), jnp.int32),
            jax.ShapeDtypeStruct((b, nf, 16 * K), jnp.float32),
        ],
    )(xyz_fine, xc_t)


CP = 32
CR = K * CP


def _sc_interp_body(cc, p_per_w, table, idxf, wf, out, idx_all, w_v0, w_v1,
                    rows_v0, rows_v1, out_v, sem_g0, sem_g1, sem_w0, sem_w1,
                    sem_out):
    wid = lax.axis_index("s") * NUM_SC + lax.axis_index("c")
    base_p0 = wid * p_per_w
    nchunk = p_per_w // CP
    rows = (rows_v0, rows_v1)
    wvs = (w_v0, w_v1)
    sgs = (sem_g0, sem_g1)
    sws = (sem_w0, sem_w1)
    pltpu.sync_copy(idxf.at[pl.ds(K * base_p0, K * p_per_w)], idx_all)

    def start(c, buf):
        pltpu.async_copy(table.at[idx_all.at[pl.ds(c * CR, CR)]], rows[buf],
                         sgs[buf])
        pltpu.async_copy(wf.at[pl.ds(base_p0 + c * CP, CP)], wvs[buf],
                         sws[buf])

    def run(c, buf):
        @pl.when(c + 1 < nchunk)
        def _():
            start(c + 1, 1 - buf)

        pltpu.make_async_copy(table.at[pl.ds(0, CR)], rows[buf],
                              sgs[buf]).wait()
        pltpu.make_async_copy(wf.at[pl.ds(0, CP)], wvs[buf], sws[buf]).wait()

        @pl.when(c > 0)
        def _():
            pltpu.make_async_copy(out_v, out.at[pl.ds(0, CP)], sem_out).wait()

        rv = rows[buf]
        wv = wvs[buf]
        for p in range(CP):
            w0 = wv[p, pl.ds(0, 16)]
            w1 = wv[p, pl.ds(16, 16)]
            w2 = wv[p, pl.ds(32, 16)]

            def col(j, _, p=p, w0=w0, w1=w1, w2=w2, rv=rv):
                s = pl.ds(j * 16, 16)
                out_v[p, s] = (w0 * rv[K * p, s] + w1 * rv[K * p + 1, s]
                               + w2 * rv[K * p + 2, s])
                return 0

            lax.fori_loop(0, cc // 16, col, 0)
        pltpu.async_copy(out_v, out.at[pl.ds(base_p0 + c * CP, CP)], sem_out)

    start(0, 0)

    def body2(g2, carry):
        run(g2 * 2, 0)
        run(g2 * 2 + 1, 1)
        return carry

    lax.fori_loop(0, nchunk // 2, body2, 0)
    pltpu.make_async_copy(out_v, out.at[pl.ds(0, CP)], sem_out).wait()


def _sc_interp(table, idx_flat, w_bcast):
    rows, cc = table.shape
    pts = idx_flat.shape[0] // K
    p_per_w = pts // NW
    mesh = plsc.VectorSubcoreMesh(core_axis_name="c", subcore_axis_name="s")
    f = functools.partial(
        pl.kernel,
        mesh=mesh,
        out_type=jax.ShapeDtypeStruct((pts, cc), jnp.float32),
        scratch_types=[
            pltpu.VMEM((K * p_per_w,), jnp.int32),
            pltpu.VMEM((CP, 16 * K), jnp.float32),
            pltpu.VMEM((CP, 16 * K), jnp.float32),
            pltpu.VMEM((CR, cc), jnp.float32),
            pltpu.VMEM((CR, cc), jnp.float32),
            pltpu.VMEM((CP, cc), jnp.float32),
            pltpu.SemaphoreType.DMA,
            pltpu.SemaphoreType.DMA,
            pltpu.SemaphoreType.DMA,
            pltpu.SemaphoreType.DMA,
            pltpu.SemaphoreType.DMA,
        ],
    )(functools.partial(_sc_interp_body, cc, p_per_w))
    return f(table, idx_flat, w_bcast)


CPG = 16


def _gn_relu(h, gamma, beta):
    n, out_ch = h.shape
    ng = out_ch // CPG
    m = (lax.broadcasted_iota(jnp.int32, (out_ch, ng), 0) // CPG ==
         lax.broadcasted_iota(jnp.int32, (out_ch, ng), 1)).astype(jnp.float32)
    mt = (lax.broadcasted_iota(jnp.int32, (ng, out_ch), 0) ==
          lax.broadcasted_iota(jnp.int32, (ng, out_ch), 1) // CPG
          ).astype(jnp.float32)
    s1 = jnp.sum(h, axis=0, keepdims=True)
    s2 = jnp.sum(h * h, axis=0, keepdims=True)
    cnt = jnp.float32(n * CPG)
    mu = jnp.dot(s1, m, preferred_element_type=jnp.float32) / cnt
    e2 = jnp.dot(s2, m, preferred_element_type=jnp.float32) / cnt
    inv = lax.rsqrt(e2 - mu * mu + EPS_GN)
    mu_c = jnp.dot(mu, mt, preferred_element_type=jnp.float32)
    inv_c = jnp.dot(inv, mt, preferred_element_type=jnp.float32)
    return jnp.maximum((h - mu_c) * inv_c * gamma + beta, 0.0)


TO = 128


def _bdot(x, w):
    return jnp.dot(x.astype(jnp.bfloat16), w.astype(jnp.bfloat16),
                   preferred_element_type=jnp.float32)


def _layer1_body(x1_ref, x2_ref, w1a_ref, w1b_ref, b1_ref, g1_ref, be1_ref,
                 out_ref):
    h = _bdot(x1_ref[0], w1a_ref[...]) + _bdot(x2_ref[0], w1b_ref[...])
    out_ref[0] = _gn_relu(h + b1_ref[...], g1_ref[...], be1_ref[...])


def _layer2_body(x_ref, w2_ref, b2_ref, g2_ref, be2_ref, out_ref):
    h = _bdot(x_ref[0], w2_ref[...])
    out_ref[0] = _gn_relu(h + b2_ref[...], g2_ref[...], be2_ref[...])


def _mlp(interp, skip, w1a_t, w1b_t, w2_t, b1, g1, be1, b2, g2, be2):
    b, nf, cc = interp.shape
    cs = skip.shape[2]
    out_ch = w2_t.shape[1]
    vec = pl.BlockSpec((1, TO), lambda i, o: (0, o))
    h1 = pl.pallas_call(
        _layer1_body,
        grid=(b, out_ch // TO),
        in_specs=[
            pl.BlockSpec((1, nf, cc), lambda i, o: (i, 0, 0)),
            pl.BlockSpec((1, nf, cs), lambda i, o: (i, 0, 0)),
            pl.BlockSpec((cc, TO), lambda i, o: (0, o)),
            pl.BlockSpec((cs, TO), lambda i, o: (0, o)),
            vec, vec, vec,
        ],
        out_specs=pl.BlockSpec((1, nf, TO), lambda i, o: (i, 0, o)),
        out_shape=jax.ShapeDtypeStruct((b, nf, out_ch), jnp.float32),
    )(interp, skip, w1a_t, w1b_t, b1, g1, be1)
    return pl.pallas_call(
        _layer2_body,
        grid=(b, out_ch // TO),
        in_specs=[
            pl.BlockSpec((1, nf, out_ch), lambda i, o: (i, 0, 0)),
            pl.BlockSpec((out_ch, TO), lambda i, o: (0, o)),
            vec, vec, vec,
        ],
        out_specs=pl.BlockSpec((1, nf, TO), lambda i, o: (i, 0, o)),
        out_shape=jax.ShapeDtypeStruct((b, nf, out_ch), jnp.float32),
    )(h1, w2_t, b2, g2, be2)


def kernel(xyz_coarse, feat_coarse, xyz_fine, feat_skip, W1, b1, g1, be1,
           W2, b2, g2, be2):
    b, nc, cc = feat_coarse.shape
    xc_t = jnp.swapaxes(xyz_coarse, 1, 2)
    idx, w = _knn(xyz_fine, xc_t)
    table = feat_coarse.reshape(b * nc, cc)
    interp = _sc_interp(table, idx.reshape(-1), w.reshape(-1, 16 * K))
    interp = interp.reshape(b, xyz_fine.shape[1], cc)
    w1a_t = W1[:, :cc].T
    w1b_t = W1[:, cc:].T
    w2_t = W2.T
    return _mlp(interp, feat_skip, w1a_t, w1b_t, w2_t,
                b1[None], g1[None], be1[None], b2[None], g2[None], be2[None])
